# baseline (speedup 1.0000x reference)
import jax
import jax.numpy as jnp
from jax.experimental import pallas as pl


def _rpn_head_kernel(xp_ref, w1_ref, b1_ref, wc_ref, bc_ref, wb_ref, bb_ref,
                     cls_ref, bbox_ref):
    xsh = [xp_ref[:, kx:kx + 32, :] for kx in range(3)]
    acc = jnp.zeros((1024, 512), dtype=jnp.float32)
    for ky in range(3):
        for kx in range(3):
            patch = xsh[kx][ky:ky + 32].reshape(1024, 512)
            acc += jnp.dot(patch, w1_ref[3 * ky + kx],
                           preferred_element_type=jnp.float32)
    h = jnp.maximum(acc + b1_ref[...], 0.0).astype(jnp.bfloat16)
    wc = wc_ref[...].astype(jnp.bfloat16).T
    wb = wb_ref[...].astype(jnp.bfloat16).T
    cls_ref[...] = (jnp.dot(h, wc, preferred_element_type=jnp.float32)
                    + bc_ref[...])
    bbox_ref[...] = (jnp.dot(h, wb, preferred_element_type=jnp.float32)
                     + bb_ref[...])


def kernel(image_features, W1, b1, Wc, bc, Wb, bb):
    x = jnp.transpose(image_features[0], (1, 2, 0))
    xp = jnp.pad(x, ((1, 1), (1, 1), (0, 0))).astype(jnp.bfloat16)
    w1 = jnp.transpose(W1, (2, 3, 1, 0)).reshape(9, 512, 512)
    w1 = w1.astype(jnp.bfloat16)

    cls, bbox = pl.pallas_call(
        _rpn_head_kernel,
        out_shape=[jax.ShapeDtypeStruct((1024, 18), jnp.float32),
                   jax.ShapeDtypeStruct((1024, 36), jnp.float32)],
    )(xp, w1, b1.reshape(1, 512), Wc.reshape(18, 512), bc.reshape(1, 18),
      Wb.reshape(36, 512), bb.reshape(1, 36))
    return (cls.reshape(9216, 2), bbox.reshape(9216, 4))
